# baseline (speedup 1.0000x reference)
import jax
import jax.numpy as jnp
from jax.experimental import pallas as pl
from jax.experimental.pallas import tpu as pltpu

_CHUNKS = (128, 128, 256, 512, 1024, 2048)


def _make_body(batch, offs):
    def body(in_hbm, out_hbm, table, rsems, wsems):
        def write(i, o, c, b):
            return pltpu.make_async_copy(
                table.at[pl.ds(o, c), :],
                out_hbm.at[b, pl.ds(o, c), :],
                wsems.at[i, b],
            )

        reads = [
            pltpu.make_async_copy(
                in_hbm.at[pl.ds(o, c), :], table.at[pl.ds(o, c), :], rsems.at[i]
            )
            for i, (o, c) in enumerate(offs)
        ]
        for r in reads:
            r.start()
        for i, (o, c) in enumerate(offs):
            reads[i].wait()
            for b in range(batch):
                write(i, o, c, b).start()
        for i, (o, c) in enumerate(offs):
            for b in range(batch):
                write(i, o, c, b).wait()

    return body


def kernel(x, pe_weight):
    batch = x.shape[0]
    max_len, d = pe_weight.shape
    offs = []
    o = 0
    for c in _CHUNKS:
        offs.append((o, c))
        o += c
    assert o == max_len
    return pl.pallas_call(
        _make_body(batch, offs),
        grid=(1,),
        in_specs=[pl.BlockSpec(memory_space=pl.ANY)],
        out_specs=pl.BlockSpec(memory_space=pl.ANY),
        out_shape=jax.ShapeDtypeStruct((batch, max_len, d), pe_weight.dtype),
        scratch_shapes=[
            pltpu.VMEM((max_len, d), pe_weight.dtype),
            pltpu.SemaphoreType.DMA((len(_CHUNKS),)),
            pltpu.SemaphoreType.DMA((len(_CHUNKS), batch)),
        ],
    )(pe_weight)


# speedup vs baseline: 1.0082x; 1.0082x over previous
import jax
from jax.experimental import pallas as pl
from jax.experimental.pallas import tpu as pltpu

_CHUNKS = (128, 128, 256, 512, 1024, 2048)


def _make_body(batch, offs):
    def body(in_hbm, out_hbm, table, rsems, wsems):
        def write(i, o, c, b):
            return pltpu.make_async_copy(
                table.at[pl.ds(o, c), :],
                out_hbm.at[b, pl.ds(o, c), :],
                wsems.at[i, b],
            )

        reads = [
            pltpu.make_async_copy(
                in_hbm.at[pl.ds(o, c), :], table.at[pl.ds(o, c), :], rsems.at[i]
            )
            for i, (o, c) in enumerate(offs)
        ]
        for r in reads:
            r.start()
        for i, (o, c) in enumerate(offs):
            reads[i].wait()
            for b in range(batch):
                write(i, o, c, b).start()
        for i, (o, c) in enumerate(offs):
            for b in range(batch):
                write(i, o, c, b).wait()

    return body


def kernel(x, pe_weight):
    batch = x.shape[0]
    max_len, d = pe_weight.shape
    offs = []
    o = 0
    for c in _CHUNKS:
        offs.append((o, c))
        o += c
    assert o == max_len
    return pl.pallas_call(
        _make_body(batch, offs),
        grid=(1,),
        in_specs=[pl.BlockSpec(memory_space=pl.ANY)],
        out_specs=pl.BlockSpec(memory_space=pl.ANY),
        out_shape=jax.ShapeDtypeStruct((batch, max_len, d), pe_weight.dtype),
        scratch_shapes=[
            pltpu.VMEM((max_len, d), pe_weight.dtype),
            pltpu.SemaphoreType.DMA((len(_CHUNKS),)),
            pltpu.SemaphoreType.DMA((len(_CHUNKS), batch)),
        ],
    )(pe_weight)
